# baseline (speedup 1.0000x reference)
import functools

import jax
import jax.numpy as jnp
from jax.experimental import pallas as pl
from jax.experimental.pallas import tpu as pltpu

B, M, E, IN, Q, OUT, H = 128, 2048, 128, 512, 512, 512, 64
BB = 8
LOGM = 11
F32 = jnp.float32


def _dg(a, b, dims):
  return jax.lax.dot_general(a, b, (dims, ((), ())),
                             preferred_element_type=F32)


def _xor_partner(x, d):
  if d < 128:
    right = pltpu.roll(x, d, 1)
    left = pltpu.roll(x, 128 - d, 1)
    lane = jax.lax.broadcasted_iota(jnp.int32, x.shape, 1)
    return jnp.where((lane & d) == 0, left, right)
  dd = d >> 7
  nrow = x.shape[0]
  right = pltpu.roll(x, dd, 0)
  left = pltpu.roll(x, nrow - dd, 0)
  row = jax.lax.broadcasted_iota(jnp.int32, x.shape, 0)
  return jnp.where((row & dd) == 0, left, right)


def _sort_record(keys, idx):
  lane = jax.lax.broadcasted_iota(jnp.int32, keys.shape, 1)
  row = jax.lax.broadcasted_iota(jnp.int32, keys.shape, 0)
  mlin = ((row & 15) << 7) | lane
  records = []
  acc = jnp.zeros(keys.shape, jnp.int32)
  bit = 0
  stages = []
  for k in range(1, LOGM + 1):
    asc = (mlin & (1 << k)) == 0
    for j in range(k - 1, -1, -1):
      d = 1 << j
      ok = _xor_partner(keys, d)
      oi = _xor_partner(idx, d)
      less = (keys < ok) | ((keys == ok) & (idx < oi))
      lower = (mlin & d) == 0
      take_self = less ^ lower ^ asc
      keys = jnp.where(take_self, keys, ok)
      idx = jnp.where(take_self, idx, oi)
      acc = acc | jnp.where(take_self, 1 << bit, 0)
      stages.append((d, len(records), bit))
      bit += 1
      if bit == 31:
        records.append(acc)
        acc = jnp.zeros(keys.shape, jnp.int32)
        bit = 0
  if bit:
    records.append(acc)
  return keys, (records, stages)


def _unsort(x, packed):
  records, stages = packed
  for d, word, bit in reversed(stages):
    take = (records[word] & (1 << bit)) != 0
    x = jnp.where(take, x, _xor_partner(x, d))
  return x


def _prefix_prod(x):
  lane = jax.lax.broadcasted_iota(jnp.int32, x.shape, 1)
  row = jax.lax.broadcasted_iota(jnp.int32, x.shape, 0)
  mlin = ((row & 15) << 7) | lane
  nrow = x.shape[0]
  for k in range(LOGM):
    s = 1 << k
    if s < 128:
      a = pltpu.roll(x, s, 1)
      b = pltpu.roll(a, 1, 0)
      sh = jnp.where(lane >= s, a, b)
    else:
      sh = pltpu.roll(x, s >> 7, 0)
    x = x * jnp.where(mlin >= s, sh, 1.0)
  return x


def _lane_allreduce(x, op):
  y = x
  w = M // 2
  while w >= 128:
    y = op(y[:, :w], y[:, w:2 * w])
    w //= 2
  s = 64
  while s >= 1:
    y = op(y, pltpu.roll(y, s, 1))
    s //= 2
  return jnp.concatenate([y] * (M // 128), axis=1)


def _softmax_lanes(x):
  e = jnp.exp(x - _lane_allreduce(x, jnp.maximum))
  return e / _lane_allreduce(e, jnp.add)


def _block_kernel(rw_ref, ww_ref, us_ref, fg_ref, inp_ref, qry_ref, mem_ref,
                  iw_ref, ib_ref, w1w_ref, w1b_ref, w2w_ref, w2b_ref,
                  r1w_ref, r1b_ref, r2w_ref, r2b_ref, ow_ref, ob_ref,
                  wr_out, w_out, u_out, y_out, rmem_out):
  inp = inp_ref[...]
  qry = qry_ref[...]
  mem = mem_ref[...]
  mem2d = mem.reshape(BB * M, E)

  e_vec = _dg(inp, iw_ref[...], ((1,), (1,))) + ib_ref[...][None, :]
  wi = _dg(e_vec, w1w_ref[...], ((1,), (1,))) + w1b_ref[...][None, :]
  cvec = _dg(wi, w2w_ref[...], ((1,), (0,)))
  cconst = jnp.sum(wi * w2b_ref[...][None, :], axis=1, keepdims=True)
  rq = _dg(qry, r1w_ref[...], ((1,), (1,))) + r1b_ref[...][None, :]
  v = _dg(rq, r2w_ref[...], ((1,), (0,)))
  rconst = jnp.sum(rq * r2b_ref[...][None, :], axis=1, keepdims=True)

  cv = jnp.concatenate([cvec, v], axis=0)
  rt = _dg(cv, mem2d, ((1,), (1,)))
  mul = jnp.concatenate(
      [rt[b:b + 1, b * M:(b + 1) * M] for b in range(BB)], axis=0)
  mul2m = jnp.concatenate(
      [rt[BB + b:BB + b + 1, b * M:(b + 1) * M] for b in range(BB)], axis=0)
  mul = mul + cconst

  w_content = _softmax_lanes(mul)

  us = 1e-05 + (1.0 - 1e-05) * us_ref[...]
  ww = ww_ref[...]
  u = us + ww - us * ww
  u = u * (1.0 - fg_ref[...] * rw_ref[...])
  u_out[...] = u

  lane = jax.lax.broadcasted_iota(jnp.int32, u.shape, 1)
  rowi = jax.lax.broadcasted_iota(jnp.int32, u.shape, 0)
  idx = ((rowi & 15) << 7) | lane
  su, recs = _sort_record(u, idx)
  alloc_sorted = (1.0 - su) * _prefix_prod(su)
  alloc = jnp.reshape(_unsort(alloc_sorted, recs), (BB, M))
  w = (w_content + alloc) * 0.5
  w_out[...] = w

  ev = jnp.sum(e_vec * v, axis=1, keepdims=True)
  mul2 = mul2m + w * ev + rconst
  wr = _softmax_lanes(mul2)
  wr_out[...] = wr

  rmem = mem + w[:, :, None] * e_vec[:, None, :]
  rmem_out[...] = rmem

  lane16k = jax.lax.broadcasted_iota(jnp.int32, (BB, BB * M), 1)
  wr_bd = jnp.where((lane16k >> LOGM) == jax.lax.broadcasted_iota(
      jnp.int32, (BB, BB * M), 0),
      jnp.concatenate([wr] * BB, axis=1), 0.0)
  outv = _dg(wr_bd, rmem.reshape(BB * M, E), ((1,), (0,)))
  y_out[...] = _dg(outv, ow_ref[...], ((1,), (1,))) + ob_ref[...][None, :]


def _full(shape):
  nd = len(shape)
  return pl.BlockSpec(shape, lambda i: (0,) * nd)


def _blk(shape):
  nd = len(shape)
  return pl.BlockSpec(shape, lambda i: (i,) + (0,) * (nd - 1))


@jax.jit
def kernel(state_vector, free_gates, input, query, memory, I_w, I_b, W1_w,
           W1_b, W2_w, W2_b, R1_w, R1_b, R2_w, R2_b, O_w, O_b):
  rw = state_vector[0].reshape(B * 16, 128)
  ww = state_vector[1].reshape(B * 16, 128)
  us = state_vector[2].reshape(B * 16, 128)
  fg = jnp.repeat(free_gates, 16, axis=0)
  inp = input[:, 0, :]
  qry = query[:, 0, :]

  grid = (B // BB,)
  out_shapes = (
      jax.ShapeDtypeStruct((B, M), jnp.float32),
      jax.ShapeDtypeStruct((B, M), jnp.float32),
      jax.ShapeDtypeStruct((B * 16, 128), jnp.float32),
      jax.ShapeDtypeStruct((B, OUT), jnp.float32),
      jax.ShapeDtypeStruct((B, M, E), jnp.float32),
  )
  in_specs = [
      _blk((BB * 16, 128)), _blk((BB * 16, 128)), _blk((BB * 16, 128)),
      _blk((BB * 16, 1)),
      _blk((BB, IN)), _blk((BB, Q)),
      _blk((BB, M, E)),
      _full((E, IN)), _full((E,)),
      _full((H, E)), _full((H,)),
      _full((H, E)), _full((H,)),
      _full((H, Q)), _full((H,)),
      _full((H, E)), _full((H,)),
      _full((OUT, E)), _full((OUT,)),
  ]
  out_specs = (
      _blk((BB, M)), _blk((BB, M)), _blk((BB * 16, 128)),
      _blk((BB, OUT)),
      _blk((BB, M, E)),
  )
  wr_o, w_o, u_o, y_o, rmem_o = pl.pallas_call(
      _block_kernel,
      grid=grid,
      in_specs=in_specs,
      out_specs=out_specs,
      out_shape=out_shapes,
  )(rw, ww, us, fg, inp, qry, memory,
    I_w, I_b, W1_w, W1_b, W2_w, W2_b, R1_w, R1_b, R2_w, R2_b, O_w, O_b)
  return (wr_o, w_o, u_o.reshape(B, M), y_o, rmem_o)


# speedup vs baseline: 1.1586x; 1.1586x over previous
import functools

import jax
import jax.numpy as jnp
from jax.experimental import pallas as pl
from jax.experimental.pallas import tpu as pltpu

B, M, E, IN, Q, OUT, H = 128, 2048, 128, 512, 512, 512, 64
BB = 8
LOGM = 11
SLICES = 2
F32 = jnp.float32


def _dg(a, b, dims):
  return jax.lax.dot_general(a, b, (dims, ((), ())),
                             preferred_element_type=F32)


def _xor_partner(x, d):
  if d < 128:
    right = pltpu.roll(x, d, 1)
    left = pltpu.roll(x, 128 - d, 1)
    lane = jax.lax.broadcasted_iota(jnp.int32, x.shape, 1)
    return jnp.where((lane & d) == 0, left, right)
  dd = d >> 7
  nrow = x.shape[0]
  right = pltpu.roll(x, dd, 0)
  left = pltpu.roll(x, nrow - dd, 0)
  row = jax.lax.broadcasted_iota(jnp.int32, x.shape, 0)
  return jnp.where((row & dd) == 0, left, right)


def _stage_list():
  stages = []
  bit = word = 0
  for k in range(1, LOGM + 1):
    for j in range(k - 1, -1, -1):
      stages.append((k, 1 << j, word, bit))
      bit += 1
      if bit == 31:
        word += 1
        bit = 0
  return stages


_STAGES = _stage_list()
_NWORDS = _STAGES[-1][2] + 1


def _sort_record(keys, idx, mref, r0):
  nrow = keys.shape[0]
  lane = jax.lax.broadcasted_iota(jnp.int32, keys.shape, 1)
  row = jax.lax.broadcasted_iota(jnp.int32, keys.shape, 0)
  mlin = ((row & 15) << 7) | lane
  acc = jnp.zeros(keys.shape, jnp.int32)
  for k, d, word, bit in _STAGES:
    asc = (mlin & (1 << k)) == 0
    ok = _xor_partner(keys, d)
    oi = _xor_partner(idx, d)
    less = (keys < ok) | ((keys == ok) & (idx < oi))
    lower = (mlin & d) == 0
    take_self = less ^ lower ^ asc
    keys = jnp.where(take_self, keys, ok)
    idx = jnp.where(take_self, idx, oi)
    acc = acc | jnp.where(take_self, 1 << bit, 0)
    if bit == 30 or (word, bit) == (_STAGES[-1][2], _STAGES[-1][3]):
      mref[word, r0:r0 + nrow, :] = acc
      acc = jnp.zeros(keys.shape, jnp.int32)
  return keys


def _unsort(x, mref, r0):
  nrow = x.shape[0]
  for _, d, word, bit in reversed(_STAGES):
    take = (mref[word, r0:r0 + nrow, :] & (1 << bit)) != 0
    x = jnp.where(take, x, _xor_partner(x, d))
  return x


def _prefix_prod(x):
  lane = jax.lax.broadcasted_iota(jnp.int32, x.shape, 1)
  row = jax.lax.broadcasted_iota(jnp.int32, x.shape, 0)
  mlin = ((row & 15) << 7) | lane
  nrow = x.shape[0]
  for k in range(LOGM):
    s = 1 << k
    if s < 128:
      a = pltpu.roll(x, s, 1)
      b = pltpu.roll(a, 1, 0)
      sh = jnp.where(lane >= s, a, b)
    else:
      sh = pltpu.roll(x, s >> 7, 0)
    x = x * jnp.where(mlin >= s, sh, 1.0)
  return x


def _lane_allreduce(x, op):
  y = x
  w = M // 2
  while w >= 128:
    y = op(y[:, :w], y[:, w:2 * w])
    w //= 2
  s = 64
  while s >= 1:
    y = op(y, pltpu.roll(y, s, 1))
    s //= 2
  return jnp.concatenate([y] * (M // 128), axis=1)


def _softmax_lanes(x):
  e = jnp.exp(x - _lane_allreduce(x, jnp.maximum))
  return e / _lane_allreduce(e, jnp.add)


def _block_kernel(rw_ref, ww_ref, us_ref, fg_ref, inp_ref, qry_ref, mem_ref,
                  iw_ref, ib_ref, w1w_ref, w1b_ref, w2w_ref, w2b_ref,
                  r1w_ref, r1b_ref, r2w_ref, r2b_ref, ow_ref, ob_ref,
                  wr_out, w_out, u_out, y_out, rmem_out, mref):
  inp = inp_ref[...]
  qry = qry_ref[...]
  mem = mem_ref[...]
  mem2d = mem.reshape(BB * M, E)

  e_vec = _dg(inp, iw_ref[...], ((1,), (1,))) + ib_ref[...][None, :]
  wi = _dg(e_vec, w1w_ref[...], ((1,), (1,))) + w1b_ref[...][None, :]
  cvec = _dg(wi, w2w_ref[...], ((1,), (0,)))
  cconst = jnp.sum(wi * w2b_ref[...][None, :], axis=1, keepdims=True)
  rq = _dg(qry, r1w_ref[...], ((1,), (1,))) + r1b_ref[...][None, :]
  v = _dg(rq, r2w_ref[...], ((1,), (0,)))
  rconst = jnp.sum(rq * r2b_ref[...][None, :], axis=1, keepdims=True)

  cv = jnp.concatenate([cvec, v], axis=0)
  rt = _dg(cv, mem2d, ((1,), (1,)))
  mul = jnp.concatenate(
      [rt[b:b + 1, b * M:(b + 1) * M] for b in range(BB)], axis=0)
  mul2m = jnp.concatenate(
      [rt[BB + b:BB + b + 1, b * M:(b + 1) * M] for b in range(BB)], axis=0)
  mul = mul + cconst

  w_content = _softmax_lanes(mul)

  us = 1e-05 + (1.0 - 1e-05) * us_ref[...]
  ww = ww_ref[...]
  u = us + ww - us * ww
  u = u * (1.0 - fg_ref[...] * rw_ref[...])
  u_out[...] = u

  allocs = []
  nsl = BB * 16 // SLICES
  for s in range(SLICES):
    usl = u[s * nsl:(s + 1) * nsl, :]
    lane = jax.lax.broadcasted_iota(jnp.int32, usl.shape, 1)
    rowi = jax.lax.broadcasted_iota(jnp.int32, usl.shape, 0)
    idx = ((rowi & 15) << 7) | lane
    su = _sort_record(usl, idx, mref, s * nsl)
    alloc_sorted = (1.0 - su) * _prefix_prod(su)
    allocs.append(_unsort(alloc_sorted, mref, s * nsl))
  alloc = jnp.reshape(jnp.concatenate(allocs, axis=0), (BB, M))
  w = (w_content + alloc) * 0.5
  w_out[...] = w

  ev = jnp.sum(e_vec * v, axis=1, keepdims=True)
  mul2 = mul2m + w * ev + rconst
  wr = _softmax_lanes(mul2)
  wr_out[...] = wr

  rmem = mem + w[:, :, None] * e_vec[:, None, :]
  rmem_out[...] = rmem

  lane16k = jax.lax.broadcasted_iota(jnp.int32, (BB, BB * M), 1)
  wr_bd = jnp.where((lane16k >> LOGM) == jax.lax.broadcasted_iota(
      jnp.int32, (BB, BB * M), 0),
      jnp.concatenate([wr] * BB, axis=1), 0.0)
  outv = _dg(wr_bd, rmem.reshape(BB * M, E), ((1,), (0,)))
  y_out[...] = _dg(outv, ow_ref[...], ((1,), (1,))) + ob_ref[...][None, :]


def _full(shape):
  nd = len(shape)
  return pl.BlockSpec(shape, lambda i: (0,) * nd)


def _blk(shape):
  nd = len(shape)
  return pl.BlockSpec(shape, lambda i: (i,) + (0,) * (nd - 1))


@jax.jit
def kernel(state_vector, free_gates, input, query, memory, I_w, I_b, W1_w,
           W1_b, W2_w, W2_b, R1_w, R1_b, R2_w, R2_b, O_w, O_b):
  rw = state_vector[0].reshape(B * 16, 128)
  ww = state_vector[1].reshape(B * 16, 128)
  us = state_vector[2].reshape(B * 16, 128)
  fg = jnp.repeat(free_gates, 16, axis=0)
  inp = input[:, 0, :]
  qry = query[:, 0, :]

  grid = (B // BB,)
  out_shapes = (
      jax.ShapeDtypeStruct((B, M), jnp.float32),
      jax.ShapeDtypeStruct((B, M), jnp.float32),
      jax.ShapeDtypeStruct((B * 16, 128), jnp.float32),
      jax.ShapeDtypeStruct((B, OUT), jnp.float32),
      jax.ShapeDtypeStruct((B, M, E), jnp.float32),
  )
  in_specs = [
      _blk((BB * 16, 128)), _blk((BB * 16, 128)), _blk((BB * 16, 128)),
      _blk((BB * 16, 1)),
      _blk((BB, IN)), _blk((BB, Q)),
      _blk((BB, M, E)),
      _full((E, IN)), _full((E,)),
      _full((H, E)), _full((H,)),
      _full((H, E)), _full((H,)),
      _full((H, Q)), _full((H,)),
      _full((H, E)), _full((H,)),
      _full((OUT, E)), _full((OUT,)),
  ]
  out_specs = (
      _blk((BB, M)), _blk((BB, M)), _blk((BB * 16, 128)),
      _blk((BB, OUT)),
      _blk((BB, M, E)),
  )
  wr_o, w_o, u_o, y_o, rmem_o = pl.pallas_call(
      _block_kernel,
      grid=grid,
      in_specs=in_specs,
      out_specs=out_specs,
      out_shape=out_shapes,
      scratch_shapes=[pltpu.VMEM((_NWORDS, BB * 16, 128), jnp.int32)],
  )(rw, ww, us, fg, inp, qry, memory,
    I_w, I_b, W1_w, W1_b, W2_w, W2_b, R1_w, R1_b, R2_w, R2_b, O_w, O_b)
  return (wr_o, w_o, u_o.reshape(B, M), y_o, rmem_o)


# speedup vs baseline: 1.2534x; 1.0818x over previous
import functools

import jax
import jax.numpy as jnp
from jax.experimental import pallas as pl
from jax.experimental.pallas import tpu as pltpu

B, M, E, IN, Q, OUT, H = 128, 2048, 128, 512, 512, 512, 64
BB = 8
LOGM = 11
SLICES = 4
F32 = jnp.float32


def _dg(a, b, dims):
  return jax.lax.dot_general(a, b, (dims, ((), ())),
                             preferred_element_type=F32)


def _xor_partner(x, d):
  if d < 128:
    right = pltpu.roll(x, d, 1)
    left = pltpu.roll(x, 128 - d, 1)
    lane = jax.lax.broadcasted_iota(jnp.int32, x.shape, 1)
    return jnp.where((lane & d) == 0, left, right)
  dd = d >> 7
  nrow = x.shape[0]
  right = pltpu.roll(x, dd, 0)
  left = pltpu.roll(x, nrow - dd, 0)
  row = jax.lax.broadcasted_iota(jnp.int32, x.shape, 0)
  return jnp.where((row & dd) == 0, left, right)


def _stage_list():
  stages = []
  bit = word = 0
  for k in range(1, LOGM + 1):
    for j in range(k - 1, -1, -1):
      stages.append((k, 1 << j, word, bit))
      bit += 1
      if bit == 31:
        word += 1
        bit = 0
  return stages


_STAGES = _stage_list()
_NWORDS = _STAGES[-1][2] + 1


def _sort_record(keys, idx, mref, r0):
  nrow = keys.shape[0]
  lane = jax.lax.broadcasted_iota(jnp.int32, keys.shape, 1)
  row = jax.lax.broadcasted_iota(jnp.int32, keys.shape, 0)
  mlin = ((row & 15) << 7) | lane
  acc = jnp.zeros(keys.shape, jnp.int32)
  for k, d, word, bit in _STAGES:
    asc = (mlin & (1 << k)) == 0
    ok = _xor_partner(keys, d)
    oi = _xor_partner(idx, d)
    less = (keys < ok) | ((keys == ok) & (idx < oi))
    lower = (mlin & d) == 0
    take_self = less ^ lower ^ asc
    keys = jnp.where(take_self, keys, ok)
    idx = jnp.where(take_self, idx, oi)
    acc = acc | jnp.where(take_self, 1 << bit, 0)
    if bit == 30 or (word, bit) == (_STAGES[-1][2], _STAGES[-1][3]):
      mref[word, r0:r0 + nrow, :] = acc
      acc = jnp.zeros(keys.shape, jnp.int32)
  return keys


def _unsort(x, mref, r0):
  nrow = x.shape[0]
  for _, d, word, bit in reversed(_STAGES):
    take = (mref[word, r0:r0 + nrow, :] & (1 << bit)) != 0
    x = jnp.where(take, x, _xor_partner(x, d))
  return x


def _prefix_prod(x):
  lane = jax.lax.broadcasted_iota(jnp.int32, x.shape, 1)
  row = jax.lax.broadcasted_iota(jnp.int32, x.shape, 0)
  mlin = ((row & 15) << 7) | lane
  nrow = x.shape[0]
  for k in range(LOGM):
    s = 1 << k
    if s < 128:
      a = pltpu.roll(x, s, 1)
      b = pltpu.roll(a, 1, 0)
      sh = jnp.where(lane >= s, a, b)
    else:
      sh = pltpu.roll(x, s >> 7, 0)
    x = x * jnp.where(mlin >= s, sh, 1.0)
  return x


def _lane_allreduce(x, op):
  y = x
  w = M // 2
  while w >= 128:
    y = op(y[:, :w], y[:, w:2 * w])
    w //= 2
  s = 64
  while s >= 1:
    y = op(y, pltpu.roll(y, s, 1))
    s //= 2
  return jnp.concatenate([y] * (M // 128), axis=1)


def _softmax_lanes(x):
  e = jnp.exp(x - _lane_allreduce(x, jnp.maximum))
  return e / _lane_allreduce(e, jnp.add)


def _block_kernel(rw_ref, ww_ref, us_ref, fg_ref, inp_ref, qry_ref, mem_ref,
                  iw_ref, ib_ref, w1w_ref, w1b_ref, w2w_ref, w2b_ref,
                  r1w_ref, r1b_ref, r2w_ref, r2b_ref, ow_ref, ob_ref,
                  wr_out, w_out, u_out, y_out, rmem_out, mref):
  inp = inp_ref[...]
  qry = qry_ref[...]
  mem = mem_ref[...]
  mem2d = mem.reshape(BB * M, E)

  e_vec = _dg(inp, iw_ref[...], ((1,), (1,))) + ib_ref[...][None, :]
  wi = _dg(e_vec, w1w_ref[...], ((1,), (1,))) + w1b_ref[...][None, :]
  cvec = _dg(wi, w2w_ref[...], ((1,), (0,)))
  cconst = jnp.sum(wi * w2b_ref[...][None, :], axis=1, keepdims=True)
  rq = _dg(qry, r1w_ref[...], ((1,), (1,))) + r1b_ref[...][None, :]
  v = _dg(rq, r2w_ref[...], ((1,), (0,)))
  rconst = jnp.sum(rq * r2b_ref[...][None, :], axis=1, keepdims=True)

  cv = jnp.concatenate([cvec, v], axis=0)
  rt = _dg(cv, mem2d, ((1,), (1,)))
  mul = jnp.concatenate(
      [rt[b:b + 1, b * M:(b + 1) * M] for b in range(BB)], axis=0)
  mul2m = jnp.concatenate(
      [rt[BB + b:BB + b + 1, b * M:(b + 1) * M] for b in range(BB)], axis=0)
  mul = mul + cconst

  w_content = _softmax_lanes(mul)

  us = 1e-05 + (1.0 - 1e-05) * us_ref[...]
  ww = ww_ref[...]
  u = us + ww - us * ww
  u = u * (1.0 - fg_ref[...] * rw_ref[...])
  u_out[...] = u

  allocs = []
  nsl = BB * 16 // SLICES
  for s in range(SLICES):
    usl = u[s * nsl:(s + 1) * nsl, :]
    lane = jax.lax.broadcasted_iota(jnp.int32, usl.shape, 1)
    rowi = jax.lax.broadcasted_iota(jnp.int32, usl.shape, 0)
    idx = ((rowi & 15) << 7) | lane
    su = _sort_record(usl, idx, mref, s * nsl)
    alloc_sorted = (1.0 - su) * _prefix_prod(su)
    allocs.append(_unsort(alloc_sorted, mref, s * nsl))
  alloc = jnp.reshape(jnp.concatenate(allocs, axis=0), (BB, M))
  w = (w_content + alloc) * 0.5
  w_out[...] = w

  ev = jnp.sum(e_vec * v, axis=1, keepdims=True)
  mul2 = mul2m + w * ev + rconst
  wr = _softmax_lanes(mul2)
  wr_out[...] = wr

  rmem = mem + w[:, :, None] * e_vec[:, None, :]
  rmem_out[...] = rmem

  lane16k = jax.lax.broadcasted_iota(jnp.int32, (BB, BB * M), 1)
  wr_bd = jnp.where((lane16k >> LOGM) == jax.lax.broadcasted_iota(
      jnp.int32, (BB, BB * M), 0),
      jnp.concatenate([wr] * BB, axis=1), 0.0)
  outv = _dg(wr_bd, rmem.reshape(BB * M, E), ((1,), (0,)))
  y_out[...] = _dg(outv, ow_ref[...], ((1,), (1,))) + ob_ref[...][None, :]


def _full(shape):
  nd = len(shape)
  return pl.BlockSpec(shape, lambda i: (0,) * nd)


def _blk(shape):
  nd = len(shape)
  return pl.BlockSpec(shape, lambda i: (i,) + (0,) * (nd - 1))


@jax.jit
def kernel(state_vector, free_gates, input, query, memory, I_w, I_b, W1_w,
           W1_b, W2_w, W2_b, R1_w, R1_b, R2_w, R2_b, O_w, O_b):
  rw = state_vector[0].reshape(B * 16, 128)
  ww = state_vector[1].reshape(B * 16, 128)
  us = state_vector[2].reshape(B * 16, 128)
  fg = jnp.repeat(free_gates, 16, axis=0)
  inp = input[:, 0, :]
  qry = query[:, 0, :]

  grid = (B // BB,)
  out_shapes = (
      jax.ShapeDtypeStruct((B, M), jnp.float32),
      jax.ShapeDtypeStruct((B, M), jnp.float32),
      jax.ShapeDtypeStruct((B * 16, 128), jnp.float32),
      jax.ShapeDtypeStruct((B, OUT), jnp.float32),
      jax.ShapeDtypeStruct((B, M, E), jnp.float32),
  )
  in_specs = [
      _blk((BB * 16, 128)), _blk((BB * 16, 128)), _blk((BB * 16, 128)),
      _blk((BB * 16, 1)),
      _blk((BB, IN)), _blk((BB, Q)),
      _blk((BB, M, E)),
      _full((E, IN)), _full((E,)),
      _full((H, E)), _full((H,)),
      _full((H, E)), _full((H,)),
      _full((H, Q)), _full((H,)),
      _full((H, E)), _full((H,)),
      _full((OUT, E)), _full((OUT,)),
  ]
  out_specs = (
      _blk((BB, M)), _blk((BB, M)), _blk((BB * 16, 128)),
      _blk((BB, OUT)),
      _blk((BB, M, E)),
  )
  wr_o, w_o, u_o, y_o, rmem_o = pl.pallas_call(
      _block_kernel,
      grid=grid,
      in_specs=in_specs,
      out_specs=out_specs,
      out_shape=out_shapes,
      scratch_shapes=[pltpu.VMEM((_NWORDS, BB * 16, 128), jnp.int32)],
  )(rw, ww, us, fg, inp, qry, memory,
    I_w, I_b, W1_w, W1_b, W2_w, W2_b, R1_w, R1_b, R2_w, R2_b, O_w, O_b)
  return (wr_o, w_o, u_o.reshape(B, M), y_o, rmem_o)


# speedup vs baseline: 1.2965x; 1.0344x over previous
import functools

import jax
import jax.numpy as jnp
from jax.experimental import pallas as pl
from jax.experimental.pallas import tpu as pltpu

B, M, E, IN, Q, OUT, H = 128, 2048, 128, 512, 512, 512, 64
BB = 8
LOGM = 11
SLICES = 8
F32 = jnp.float32


def _dg(a, b, dims):
  return jax.lax.dot_general(a, b, (dims, ((), ())),
                             preferred_element_type=F32)


def _xor_partner(x, d):
  if d < 128:
    right = pltpu.roll(x, d, 1)
    left = pltpu.roll(x, 128 - d, 1)
    lane = jax.lax.broadcasted_iota(jnp.int32, x.shape, 1)
    return jnp.where((lane & d) == 0, left, right)
  dd = d >> 7
  nrow = x.shape[0]
  right = pltpu.roll(x, dd, 0)
  left = pltpu.roll(x, nrow - dd, 0)
  row = jax.lax.broadcasted_iota(jnp.int32, x.shape, 0)
  return jnp.where((row & dd) == 0, left, right)


def _stage_list():
  stages = []
  bit = word = 0
  for k in range(1, LOGM + 1):
    for j in range(k - 1, -1, -1):
      stages.append((k, 1 << j, word, bit))
      bit += 1
      if bit == 31:
        word += 1
        bit = 0
  return stages


_STAGES = _stage_list()
_NWORDS = _STAGES[-1][2] + 1


def _sort_record(keys, idx, mref, r0):
  nrow = keys.shape[0]
  lane = jax.lax.broadcasted_iota(jnp.int32, keys.shape, 1)
  row = jax.lax.broadcasted_iota(jnp.int32, keys.shape, 0)
  mlin = ((row & 15) << 7) | lane
  acc = jnp.zeros(keys.shape, jnp.int32)
  for k, d, word, bit in _STAGES:
    asc = (mlin & (1 << k)) == 0
    ok = _xor_partner(keys, d)
    oi = _xor_partner(idx, d)
    less = (keys < ok) | ((keys == ok) & (idx < oi))
    lower = (mlin & d) == 0
    take_self = less ^ lower ^ asc
    keys = jnp.where(take_self, keys, ok)
    idx = jnp.where(take_self, idx, oi)
    acc = acc | jnp.where(take_self, 1 << bit, 0)
    if bit == 30 or (word, bit) == (_STAGES[-1][2], _STAGES[-1][3]):
      mref[word, r0:r0 + nrow, :] = acc
      acc = jnp.zeros(keys.shape, jnp.int32)
  return keys


def _unsort(x, mref, r0):
  nrow = x.shape[0]
  for _, d, word, bit in reversed(_STAGES):
    take = (mref[word, r0:r0 + nrow, :] & (1 << bit)) != 0
    x = jnp.where(take, x, _xor_partner(x, d))
  return x


def _prefix_prod(x):
  lane = jax.lax.broadcasted_iota(jnp.int32, x.shape, 1)
  row = jax.lax.broadcasted_iota(jnp.int32, x.shape, 0)
  mlin = ((row & 15) << 7) | lane
  nrow = x.shape[0]
  for k in range(LOGM):
    s = 1 << k
    if s < 128:
      a = pltpu.roll(x, s, 1)
      b = pltpu.roll(a, 1, 0)
      sh = jnp.where(lane >= s, a, b)
    else:
      sh = pltpu.roll(x, s >> 7, 0)
    x = x * jnp.where(mlin >= s, sh, 1.0)
  return x


def _lane_allreduce(x, op):
  y = x
  w = M // 2
  while w >= 128:
    y = op(y[:, :w], y[:, w:2 * w])
    w //= 2
  s = 64
  while s >= 1:
    y = op(y, pltpu.roll(y, s, 1))
    s //= 2
  return jnp.concatenate([y] * (M // 128), axis=1)


def _softmax_lanes(x):
  e = jnp.exp(x - _lane_allreduce(x, jnp.maximum))
  return e / _lane_allreduce(e, jnp.add)


def _block_kernel(rw_ref, ww_ref, us_ref, fg_ref, inp_ref, qry_ref, mem_ref,
                  iw_ref, ib_ref, w1w_ref, w1b_ref, w2w_ref, w2b_ref,
                  r1w_ref, r1b_ref, r2w_ref, r2b_ref, ow_ref, ob_ref,
                  wr_out, w_out, u_out, y_out, rmem_out, mref):
  inp = inp_ref[...]
  qry = qry_ref[...]
  mem = mem_ref[...]
  mem2d = mem.reshape(BB * M, E)

  e_vec = _dg(inp, iw_ref[...], ((1,), (1,))) + ib_ref[...][None, :]
  wi = _dg(e_vec, w1w_ref[...], ((1,), (1,))) + w1b_ref[...][None, :]
  cvec = _dg(wi, w2w_ref[...], ((1,), (0,)))
  cconst = jnp.sum(wi * w2b_ref[...][None, :], axis=1, keepdims=True)
  rq = _dg(qry, r1w_ref[...], ((1,), (1,))) + r1b_ref[...][None, :]
  v = _dg(rq, r2w_ref[...], ((1,), (0,)))
  rconst = jnp.sum(rq * r2b_ref[...][None, :], axis=1, keepdims=True)

  cv = jnp.concatenate([cvec, v], axis=0)
  rt = _dg(cv, mem2d, ((1,), (1,)))
  mul = jnp.concatenate(
      [rt[b:b + 1, b * M:(b + 1) * M] for b in range(BB)], axis=0)
  mul2m = jnp.concatenate(
      [rt[BB + b:BB + b + 1, b * M:(b + 1) * M] for b in range(BB)], axis=0)
  mul = mul + cconst

  w_content = _softmax_lanes(mul)

  us = 1e-05 + (1.0 - 1e-05) * us_ref[...]
  ww = ww_ref[...]
  u = us + ww - us * ww
  u = u * (1.0 - fg_ref[...] * rw_ref[...])
  u_out[...] = u

  allocs = []
  nsl = BB * 16 // SLICES
  for s in range(SLICES):
    usl = u[s * nsl:(s + 1) * nsl, :]
    lane = jax.lax.broadcasted_iota(jnp.int32, usl.shape, 1)
    rowi = jax.lax.broadcasted_iota(jnp.int32, usl.shape, 0)
    idx = ((rowi & 15) << 7) | lane
    su = _sort_record(usl, idx, mref, s * nsl)
    alloc_sorted = (1.0 - su) * _prefix_prod(su)
    allocs.append(_unsort(alloc_sorted, mref, s * nsl))
  alloc = jnp.reshape(jnp.concatenate(allocs, axis=0), (BB, M))
  w = (w_content + alloc) * 0.5
  w_out[...] = w

  ev = jnp.sum(e_vec * v, axis=1, keepdims=True)
  mul2 = mul2m + w * ev + rconst
  wr = _softmax_lanes(mul2)
  wr_out[...] = wr

  rmem = mem + w[:, :, None] * e_vec[:, None, :]
  rmem_out[...] = rmem

  lane16k = jax.lax.broadcasted_iota(jnp.int32, (BB, BB * M), 1)
  wr_bd = jnp.where((lane16k >> LOGM) == jax.lax.broadcasted_iota(
      jnp.int32, (BB, BB * M), 0),
      jnp.concatenate([wr] * BB, axis=1), 0.0)
  outv = _dg(wr_bd, rmem.reshape(BB * M, E), ((1,), (0,)))
  y_out[...] = _dg(outv, ow_ref[...], ((1,), (1,))) + ob_ref[...][None, :]


def _full(shape):
  nd = len(shape)
  return pl.BlockSpec(shape, lambda i: (0,) * nd)


def _blk(shape):
  nd = len(shape)
  return pl.BlockSpec(shape, lambda i: (i,) + (0,) * (nd - 1))


@jax.jit
def kernel(state_vector, free_gates, input, query, memory, I_w, I_b, W1_w,
           W1_b, W2_w, W2_b, R1_w, R1_b, R2_w, R2_b, O_w, O_b):
  rw = state_vector[0].reshape(B * 16, 128)
  ww = state_vector[1].reshape(B * 16, 128)
  us = state_vector[2].reshape(B * 16, 128)
  fg = jnp.repeat(free_gates, 16, axis=0)
  inp = input[:, 0, :]
  qry = query[:, 0, :]

  grid = (B // BB,)
  out_shapes = (
      jax.ShapeDtypeStruct((B, M), jnp.float32),
      jax.ShapeDtypeStruct((B, M), jnp.float32),
      jax.ShapeDtypeStruct((B * 16, 128), jnp.float32),
      jax.ShapeDtypeStruct((B, OUT), jnp.float32),
      jax.ShapeDtypeStruct((B, M, E), jnp.float32),
  )
  in_specs = [
      _blk((BB * 16, 128)), _blk((BB * 16, 128)), _blk((BB * 16, 128)),
      _blk((BB * 16, 1)),
      _blk((BB, IN)), _blk((BB, Q)),
      _blk((BB, M, E)),
      _full((E, IN)), _full((E,)),
      _full((H, E)), _full((H,)),
      _full((H, E)), _full((H,)),
      _full((H, Q)), _full((H,)),
      _full((H, E)), _full((H,)),
      _full((OUT, E)), _full((OUT,)),
  ]
  out_specs = (
      _blk((BB, M)), _blk((BB, M)), _blk((BB * 16, 128)),
      _blk((BB, OUT)),
      _blk((BB, M, E)),
  )
  wr_o, w_o, u_o, y_o, rmem_o = pl.pallas_call(
      _block_kernel,
      grid=grid,
      in_specs=in_specs,
      out_specs=out_specs,
      out_shape=out_shapes,
      scratch_shapes=[pltpu.VMEM((_NWORDS, BB * 16, 128), jnp.int32)],
  )(rw, ww, us, fg, inp, qry, memory,
    I_w, I_b, W1_w, W1_b, W2_w, W2_b, R1_w, R1_b, R2_w, R2_b, O_w, O_b)
  return (wr_o, w_o, u_o.reshape(B, M), y_o, rmem_o)


# speedup vs baseline: 1.3565x; 1.0463x over previous
import functools

import jax
import jax.numpy as jnp
from jax.experimental import pallas as pl
from jax.experimental.pallas import tpu as pltpu

B, M, E, IN, Q, OUT, H = 128, 2048, 128, 512, 512, 512, 64
BB = 8
LOGM = 11
SLICES = 16
F32 = jnp.float32


def _dg(a, b, dims):
  return jax.lax.dot_general(a, b, (dims, ((), ())),
                             preferred_element_type=F32)


def _xor_partner(x, d):
  if d < 128:
    right = pltpu.roll(x, d, 1)
    left = pltpu.roll(x, 128 - d, 1)
    lane = jax.lax.broadcasted_iota(jnp.int32, x.shape, 1)
    return jnp.where((lane & d) == 0, left, right)
  dd = d >> 7
  nrow = x.shape[0]
  right = pltpu.roll(x, dd, 0)
  left = pltpu.roll(x, nrow - dd, 0)
  row = jax.lax.broadcasted_iota(jnp.int32, x.shape, 0)
  return jnp.where((row & dd) == 0, left, right)


def _stage_list():
  stages = []
  bit = word = 0
  for k in range(1, LOGM + 1):
    for j in range(k - 1, -1, -1):
      stages.append((k, 1 << j, word, bit))
      bit += 1
      if bit == 31:
        word += 1
        bit = 0
  return stages


_STAGES = _stage_list()
_NWORDS = _STAGES[-1][2] + 1


def _sort_record(keys, idx, mref, r0):
  nrow = keys.shape[0]
  lane = jax.lax.broadcasted_iota(jnp.int32, keys.shape, 1)
  row = jax.lax.broadcasted_iota(jnp.int32, keys.shape, 0)
  mlin = ((row & 15) << 7) | lane
  acc = jnp.zeros(keys.shape, jnp.int32)
  for k, d, word, bit in _STAGES:
    asc = (mlin & (1 << k)) == 0
    ok = _xor_partner(keys, d)
    oi = _xor_partner(idx, d)
    less = (keys < ok) | ((keys == ok) & (idx < oi))
    lower = (mlin & d) == 0
    take_self = less ^ lower ^ asc
    keys = jnp.where(take_self, keys, ok)
    idx = jnp.where(take_self, idx, oi)
    acc = acc | jnp.where(take_self, 1 << bit, 0)
    if bit == 30 or (word, bit) == (_STAGES[-1][2], _STAGES[-1][3]):
      mref[word, r0:r0 + nrow, :] = acc
      acc = jnp.zeros(keys.shape, jnp.int32)
  return keys


def _unsort(x, mref, r0):
  nrow = x.shape[0]
  for _, d, word, bit in reversed(_STAGES):
    take = (mref[word, r0:r0 + nrow, :] & (1 << bit)) != 0
    x = jnp.where(take, x, _xor_partner(x, d))
  return x


def _prefix_prod(x):
  lane = jax.lax.broadcasted_iota(jnp.int32, x.shape, 1)
  row = jax.lax.broadcasted_iota(jnp.int32, x.shape, 0)
  mlin = ((row & 15) << 7) | lane
  nrow = x.shape[0]
  for k in range(LOGM):
    s = 1 << k
    if s < 128:
      a = pltpu.roll(x, s, 1)
      b = pltpu.roll(a, 1, 0)
      sh = jnp.where(lane >= s, a, b)
    else:
      sh = pltpu.roll(x, s >> 7, 0)
    x = x * jnp.where(mlin >= s, sh, 1.0)
  return x


def _lane_allreduce(x, op):
  y = x
  w = M // 2
  while w >= 128:
    y = op(y[:, :w], y[:, w:2 * w])
    w //= 2
  s = 64
  while s >= 1:
    y = op(y, pltpu.roll(y, s, 1))
    s //= 2
  return jnp.concatenate([y] * (M // 128), axis=1)


def _softmax_lanes(x):
  e = jnp.exp(x - _lane_allreduce(x, jnp.maximum))
  return e / _lane_allreduce(e, jnp.add)


def _block_kernel(rw_ref, ww_ref, us_ref, fg_ref, inp_ref, qry_ref, mem_ref,
                  iw_ref, ib_ref, w1w_ref, w1b_ref, w2w_ref, w2b_ref,
                  r1w_ref, r1b_ref, r2w_ref, r2b_ref, ow_ref, ob_ref,
                  wr_out, w_out, u_out, y_out, rmem_out, mref):
  inp = inp_ref[...]
  qry = qry_ref[...]
  mem = mem_ref[...]
  mem2d = mem.reshape(BB * M, E)

  e_vec = _dg(inp, iw_ref[...], ((1,), (1,))) + ib_ref[...][None, :]
  wi = _dg(e_vec, w1w_ref[...], ((1,), (1,))) + w1b_ref[...][None, :]
  cvec = _dg(wi, w2w_ref[...], ((1,), (0,)))
  cconst = jnp.sum(wi * w2b_ref[...][None, :], axis=1, keepdims=True)
  rq = _dg(qry, r1w_ref[...], ((1,), (1,))) + r1b_ref[...][None, :]
  v = _dg(rq, r2w_ref[...], ((1,), (0,)))
  rconst = jnp.sum(rq * r2b_ref[...][None, :], axis=1, keepdims=True)

  cv = jnp.concatenate([cvec, v], axis=0)
  rt = _dg(cv, mem2d, ((1,), (1,)))
  mul = jnp.concatenate(
      [rt[b:b + 1, b * M:(b + 1) * M] for b in range(BB)], axis=0)
  mul2m = jnp.concatenate(
      [rt[BB + b:BB + b + 1, b * M:(b + 1) * M] for b in range(BB)], axis=0)
  mul = mul + cconst

  w_content = _softmax_lanes(mul)

  us = 1e-05 + (1.0 - 1e-05) * us_ref[...]
  ww = ww_ref[...]
  u = us + ww - us * ww
  u = u * (1.0 - fg_ref[...] * rw_ref[...])
  u_out[...] = u

  allocs = []
  nsl = BB * 16 // SLICES
  for s in range(SLICES):
    usl = u[s * nsl:(s + 1) * nsl, :]
    lane = jax.lax.broadcasted_iota(jnp.int32, usl.shape, 1)
    rowi = jax.lax.broadcasted_iota(jnp.int32, usl.shape, 0)
    idx = ((rowi & 15) << 7) | lane
    su = _sort_record(usl, idx, mref, s * nsl)
    alloc_sorted = (1.0 - su) * _prefix_prod(su)
    allocs.append(_unsort(alloc_sorted, mref, s * nsl))
  alloc = jnp.reshape(jnp.concatenate(allocs, axis=0), (BB, M))
  w = (w_content + alloc) * 0.5
  w_out[...] = w

  ev = jnp.sum(e_vec * v, axis=1, keepdims=True)
  mul2 = mul2m + w * ev + rconst
  wr = _softmax_lanes(mul2)
  wr_out[...] = wr

  rmem = mem + w[:, :, None] * e_vec[:, None, :]
  rmem_out[...] = rmem

  lane16k = jax.lax.broadcasted_iota(jnp.int32, (BB, BB * M), 1)
  wr_bd = jnp.where((lane16k >> LOGM) == jax.lax.broadcasted_iota(
      jnp.int32, (BB, BB * M), 0),
      jnp.concatenate([wr] * BB, axis=1), 0.0)
  outv = _dg(wr_bd, rmem.reshape(BB * M, E), ((1,), (0,)))
  y_out[...] = _dg(outv, ow_ref[...], ((1,), (1,))) + ob_ref[...][None, :]


def _full(shape):
  nd = len(shape)
  return pl.BlockSpec(shape, lambda i: (0,) * nd)


def _blk(shape):
  nd = len(shape)
  return pl.BlockSpec(shape, lambda i: (i,) + (0,) * (nd - 1))


@jax.jit
def kernel(state_vector, free_gates, input, query, memory, I_w, I_b, W1_w,
           W1_b, W2_w, W2_b, R1_w, R1_b, R2_w, R2_b, O_w, O_b):
  rw = state_vector[0].reshape(B * 16, 128)
  ww = state_vector[1].reshape(B * 16, 128)
  us = state_vector[2].reshape(B * 16, 128)
  fg = jnp.repeat(free_gates, 16, axis=0)
  inp = input[:, 0, :]
  qry = query[:, 0, :]

  grid = (B // BB,)
  out_shapes = (
      jax.ShapeDtypeStruct((B, M), jnp.float32),
      jax.ShapeDtypeStruct((B, M), jnp.float32),
      jax.ShapeDtypeStruct((B * 16, 128), jnp.float32),
      jax.ShapeDtypeStruct((B, OUT), jnp.float32),
      jax.ShapeDtypeStruct((B, M, E), jnp.float32),
  )
  in_specs = [
      _blk((BB * 16, 128)), _blk((BB * 16, 128)), _blk((BB * 16, 128)),
      _blk((BB * 16, 1)),
      _blk((BB, IN)), _blk((BB, Q)),
      _blk((BB, M, E)),
      _full((E, IN)), _full((E,)),
      _full((H, E)), _full((H,)),
      _full((H, E)), _full((H,)),
      _full((H, Q)), _full((H,)),
      _full((H, E)), _full((H,)),
      _full((OUT, E)), _full((OUT,)),
  ]
  out_specs = (
      _blk((BB, M)), _blk((BB, M)), _blk((BB * 16, 128)),
      _blk((BB, OUT)),
      _blk((BB, M, E)),
  )
  wr_o, w_o, u_o, y_o, rmem_o = pl.pallas_call(
      _block_kernel,
      grid=grid,
      in_specs=in_specs,
      out_specs=out_specs,
      out_shape=out_shapes,
      scratch_shapes=[pltpu.VMEM((_NWORDS, BB * 16, 128), jnp.int32)],
  )(rw, ww, us, fg, inp, qry, memory,
    I_w, I_b, W1_w, W1_b, W2_w, W2_b, R1_w, R1_b, R2_w, R2_b, O_w, O_b)
  return (wr_o, w_o, u_o.reshape(B, M), y_o, rmem_o)
